# baseline (speedup 1.0000x reference)
import jax
import jax.numpy as jnp
from jax.experimental import pallas as pl
from jax.experimental.pallas import tpu as pltpu


def _blank_body(cam_ref, imgs_ref, masks_ref, imgs_out_ref, masks_out_ref):
    p = pl.program_id(0)
    b = p // 6
    c = p % 6
    keep = jnp.where(cam_ref[b] == c, 0.0, 1.0).astype(jnp.float32)
    imgs_out_ref[...] = imgs_ref[...] * keep
    masks_out_ref[...] = masks_ref[...] * keep


def kernel(imgs, grids, masks, cam_choice):
    B, NC, C, H, W = imgs.shape
    n_img = C * H * W
    n_msk = H * W
    imgs2 = imgs.reshape(B * NC, 384, 1152)
    masks2 = masks.reshape(B * NC, 128, 1152)

    imgs_out, masks_out = pl.pallas_call(
        _blank_body,
        grid_spec=pltpu.PrefetchScalarGridSpec(
            num_scalar_prefetch=1,
            grid=(B * NC,),
            in_specs=[
                pl.BlockSpec((1, 384, 1152), lambda p, cam: (p, 0, 0)),
                pl.BlockSpec((1, 128, 1152), lambda p, cam: (p, 0, 0)),
            ],
            out_specs=[
                pl.BlockSpec((1, 384, 1152), lambda p, cam: (p, 0, 0)),
                pl.BlockSpec((1, 128, 1152), lambda p, cam: (p, 0, 0)),
            ],
        ),
        out_shape=[
            jax.ShapeDtypeStruct(imgs2.shape, imgs2.dtype),
            jax.ShapeDtypeStruct(masks2.shape, masks2.dtype),
        ],
    )(cam_choice.astype(jnp.int32), imgs2, masks2)

    return (imgs_out.reshape(imgs.shape), grids, masks_out.reshape(masks.shape))
